# baseline (speedup 1.0000x reference)
import functools

import jax
import jax.numpy as jnp
from jax.experimental import pallas as pl
from jax.experimental.pallas import tpu as pltpu

_T = 16384
_D = 2048
_E = 16
_BT = 1024


def _router_body(x_ref, w_ref, w_out_ref, e_out_ref):
    logits = jnp.dot(x_ref[...], w_ref[...], preferred_element_type=jnp.float32)
    col = jax.lax.broadcasted_iota(jnp.int32, logits.shape, 1)
    m1 = jnp.max(logits, axis=1, keepdims=True)
    a1 = jnp.argmax(logits, axis=1)[:, None]
    masked = jnp.where(col == a1, -jnp.inf, logits)
    m2 = jnp.max(masked, axis=1, keepdims=True)
    a2 = jnp.argmax(masked, axis=1)[:, None]
    e = jnp.exp(m2 - m1)
    s = 1.0 / (1.0 + e)
    w_out_ref[...] = jnp.concatenate([s, e * s], axis=1)
    e_out_ref[...] = jnp.concatenate([a1, a2], axis=1)


@jax.jit
def kernel(x_TD, kernel_DE):
    x_TD = jnp.asarray(x_TD, jnp.float32)
    grid = (_T // _BT,)
    weights, experts = pl.pallas_call(
        _router_body,
        grid=grid,
        in_specs=[
            pl.BlockSpec((_BT, _D), lambda i: (i, 0)),
            pl.BlockSpec((_D, _E), lambda i: (0, 0)),
        ],
        out_specs=[
            pl.BlockSpec((_BT, 2), lambda i: (i, 0)),
            pl.BlockSpec((_BT, 2), lambda i: (i, 0)),
        ],
        out_shape=[
            jax.ShapeDtypeStruct((_T, 2), jnp.float32),
            jax.ShapeDtypeStruct((_T, 2), jnp.int32),
        ],
        compiler_params=pltpu.CompilerParams(
            dimension_semantics=("arbitrary",),
            vmem_limit_bytes=100 * 1024 * 1024,
        ),
    )(x_TD, kernel_DE)
    return (weights, experts)


# speedup vs baseline: 1.3785x; 1.3785x over previous
import jax
import jax.numpy as jnp
from jax import lax
from jax.experimental import pallas as pl
from jax.experimental.pallas import tpu as pltpu

_T = 16384
_D = 2048
_E = 16
_BT = 2048


def _router_body(x_ref, w_ref, wo_ref, eo_ref):
    logitsT = lax.dot_general(
        w_ref[...], x_ref[...], (((0,), (1,)), ((), ())),
        preferred_element_type=jnp.float32,
    )
    row = lax.broadcasted_iota(jnp.int32, (_E, _BT), 0)
    m1 = jnp.max(logitsT, axis=0, keepdims=True)
    a1 = jnp.argmax(logitsT, axis=0)[None, :]
    masked = jnp.where(row == a1, jnp.float32(-jnp.inf), logitsT)
    m2 = jnp.max(masked, axis=0, keepdims=True)
    a2 = jnp.argmax(masked, axis=0)[None, :]
    e = jnp.exp(m2 - m1)
    s = 1.0 / (1.0 + e)
    wo_ref[...] = jnp.concatenate([s, e * s], axis=0)
    eo_ref[...] = jnp.concatenate([a1, a2], axis=0)


@jax.jit
def kernel(x_TD, kernel_DE):
    x_TD = jnp.asarray(x_TD, jnp.float32)
    wo, eo = pl.pallas_call(
        _router_body,
        grid=(_T // _BT,),
        in_specs=[
            pl.BlockSpec((_BT, _D), lambda i: (i, 0)),
            pl.BlockSpec((_D, _E), lambda i: (0, 0)),
        ],
        out_specs=[
            pl.BlockSpec((2, _BT), lambda i: (0, i)),
            pl.BlockSpec((2, _BT), lambda i: (0, i)),
        ],
        out_shape=[
            jax.ShapeDtypeStruct((2, _T), jnp.float32),
            jax.ShapeDtypeStruct((2, _T), jnp.int32),
        ],
        compiler_params=pltpu.CompilerParams(vmem_limit_bytes=100 * 1024 * 1024),
    )(x_TD, kernel_DE)
    return (wo.T, eo.T)


# speedup vs baseline: 1.4379x; 1.0431x over previous
import jax
import jax.numpy as jnp
from jax import lax
from jax.experimental import pallas as pl
from jax.experimental.pallas import tpu as pltpu

_T = 16384
_D = 2048
_E = 16
_BT = 1024


def _router_body(x_ref, w_ref, wo_ref, eo_ref):
    logitsT = lax.dot_general(
        w_ref[...], x_ref[...], (((0,), (1,)), ((), ())),
        preferred_element_type=jnp.float32,
    )
    row = lax.broadcasted_iota(jnp.int32, (_E, _BT), 0)
    m1 = jnp.max(logitsT, axis=0, keepdims=True)
    a1 = jnp.argmax(logitsT, axis=0)[None, :]
    masked = jnp.where(row == a1, jnp.float32(-jnp.inf), logitsT)
    m2 = jnp.max(masked, axis=0, keepdims=True)
    a2 = jnp.argmax(masked, axis=0)[None, :]
    e = jnp.exp(m2 - m1)
    s = 1.0 / (1.0 + e)
    wo_ref[...] = jnp.concatenate([s, e * s], axis=0)
    eo_ref[...] = jnp.concatenate([a1, a2], axis=0)


@jax.jit
def kernel(x_TD, kernel_DE):
    x_TD = jnp.asarray(x_TD, jnp.float32)
    wo, eo = pl.pallas_call(
        _router_body,
        grid=(_T // _BT,),
        in_specs=[
            pl.BlockSpec((_BT, _D), lambda i: (i, 0)),
            pl.BlockSpec((_D, _E), lambda i: (0, 0)),
        ],
        out_specs=[
            pl.BlockSpec((2, _BT), lambda i: (0, i)),
            pl.BlockSpec((2, _BT), lambda i: (0, i)),
        ],
        out_shape=[
            jax.ShapeDtypeStruct((2, _T), jnp.float32),
            jax.ShapeDtypeStruct((2, _T), jnp.int32),
        ],
        compiler_params=pltpu.CompilerParams(vmem_limit_bytes=100 * 1024 * 1024),
    )(x_TD, kernel_DE)
    return (wo.T, eo.T)
